# baseline (speedup 1.0000x reference)
import functools

import jax
import jax.numpy as jnp
from jax import lax
from jax.experimental import pallas as pl
from jax.experimental.pallas import tpu as pltpu
from jax.experimental.pallas import tpu_sc as plsc

N = 10000
E = 320000
D = 128
E_DIM = 16
EPS = 1e-5

NC = 2
NS = 16
NW = NC * NS

C = 128
CHUNKS = E // C

CA = 80
TILE_E = E // NS
NCH = TILE_E // CA
RING = 4

ROWS_A = 624
ROWS_LAST = N - (NS - 1) * ROWS_A

@functools.cache
def _mesh():
    return plsc.VectorSubcoreMesh(
        core_axis_name="c", subcore_axis_name="s", num_cores=NC, num_subcores=NS)


def _sc_deg_body(col_hbm, zeros_n, deg_out, cbuf, deg_local, sem):
    c = lax.axis_index("c")
    s = lax.axis_index("s")
    wid = s * NC + c
    pltpu.sync_copy(zeros_n, deg_local)
    nbase = CHUNKS // NW
    n_i = nbase + jnp.where(wid < CHUNKS % NW, 1, 0)
    ones = jnp.full((16,), 1.0, jnp.float32)

    @pl.loop(0, n_i)
    def _(i):
        base = (wid + i * NW) * C
        pltpu.sync_copy(col_hbm.at[pl.ds(base, C)], cbuf)
        for j in range(C // 16):
            idx = cbuf[pl.ds(j * 16, 16)]
            plsc.addupdate_scatter(deg_local, [idx], ones)

    pltpu.sync_copy(deg_local, deg_out.at[wid].at[0])


@functools.cache
def _sc_deg_kernel():
    return pl.kernel(
        _sc_deg_body,
        out_type=jax.ShapeDtypeStruct((NW, 1, N), jnp.float32),
        mesh=_mesh(),
        compiler_params=pltpu.CompilerParams(needs_layout_passes=False),
        scratch_types=[
            pltpu.VMEM((C,), jnp.int32),
            pltpu.VMEM((N,), jnp.float32),
            pltpu.SemaphoreType.DMA,
        ],
    )


def _sc_deg(col, zeros_n):
    return _sc_deg_kernel()(col, zeros_n)


def _sc_agg_body(row_hbm, col_hbm, hh_hbm, t_hbm, zrows,
                 a1_out, a2_out, rbuf, cbuf, pbuf, acc, *sems):
    c = lax.axis_index("c")
    s = lax.axis_index("s")
    r0 = s * ROWS_A
    seml = sems[:RING]
    sems_ = sems[RING:2 * RING]
    semg = sems[2 * RING:]
    base_t = s * TILE_E

    def issue_loads(j, b):
        off = base_t + j * CA
        pltpu.async_copy(col_hbm.at[pl.ds(off, CA)], cbuf.at[b], seml[b])

        @pl.when(c == 0)
        def _():
            pltpu.async_copy(row_hbm.at[pl.ds(off, CA)], rbuf.at[b], seml[b])

        @pl.when(c == 1)
        def _():
            pltpu.async_copy(t_hbm.at[pl.ds(off, CA)], pbuf.at[b], seml[b])

    def wait_loads(j, b):
        off = base_t + j * CA
        pltpu.make_async_copy(
            col_hbm.at[pl.ds(off, CA)], cbuf.at[b], seml[b]).wait()

        @pl.when(c == 0)
        def _():
            pltpu.make_async_copy(
                row_hbm.at[pl.ds(off, CA)], rbuf.at[b], seml[b]).wait()

        @pl.when(c == 1)
        def _():
            pltpu.make_async_copy(
                t_hbm.at[pl.ds(off, CA)], pbuf.at[b], seml[b]).wait()

    def issue_scatter(b):
        pltpu.async_copy(pbuf.at[b], acc.at[cbuf.at[b]], sems_[b], add=True)

    def wait_scatter(b):
        pltpu.make_async_copy(pbuf.at[b], acc.at[cbuf.at[b]], sems_[b]).wait()

    def issue_gather(b):
        pltpu.async_copy(hh_hbm.at[rbuf.at[b]], pbuf.at[b], semg[b])

    def wait_gather(b):
        pltpu.make_async_copy(hh_hbm.at[rbuf.at[b]], pbuf.at[b], semg[b]).wait()

    @pl.when(s < NS - 1)
    def _():
        pltpu.sync_copy(zrows.at[pl.ds(0, ROWS_A)], acc.at[pl.ds(r0, ROWS_A)])

    @pl.when(s == NS - 1)
    def _():
        pltpu.sync_copy(zrows, acc.at[pl.ds(r0, ROWS_LAST)])

    issue_loads(0, 0)
    issue_loads(1, 1)

    @pl.when(c == 0)
    def _():
        wait_loads(0, 0)
        issue_gather(0)

    plsc.subcore_barrier()

    @pl.loop(0, (NCH + RING - 1) // RING)
    def _(p):
        for q in range(RING):
            b = q
            bn = (q + 1) % RING
            b2 = (q + 2) % RING
            j = p * RING + q

            @pl.when(j < NCH)
            def _():
                @pl.when(c == 1)
                def _():
                    wait_loads(j, b)

                @pl.when(c == 0)
                def _():
                    wait_gather(b)

                @pl.when(j >= 2)
                def _():
                    wait_scatter(b2)

                @pl.when(j + 2 < NCH)
                def _():
                    issue_loads(j + 2, b2)

                issue_scatter(b)

                @pl.when((c == 0) & (j + 1 < NCH))
                def _():
                    wait_loads(j + 1, bn)
                    issue_gather(bn)

    wait_scatter((NCH - 2) % RING)
    wait_scatter((NCH - 1) % RING)
    plsc.subcore_barrier()

    @pl.when((c == 0) & (s < NS - 1))
    def _():
        pltpu.sync_copy(acc.at[pl.ds(r0, ROWS_A)], a1_out.at[pl.ds(r0, ROWS_A)])

    @pl.when((c == 0) & (s == NS - 1))
    def _():
        pltpu.sync_copy(acc.at[pl.ds(r0, ROWS_LAST)],
                        a1_out.at[pl.ds(r0, ROWS_LAST)])

    @pl.when((c == 1) & (s < NS - 1))
    def _():
        pltpu.sync_copy(acc.at[pl.ds(r0, ROWS_A)], a2_out.at[pl.ds(r0, ROWS_A)])

    @pl.when((c == 1) & (s == NS - 1))
    def _():
        pltpu.sync_copy(acc.at[pl.ds(r0, ROWS_LAST)],
                        a2_out.at[pl.ds(r0, ROWS_LAST)])


@functools.cache
def _sc_agg_kernel():
    return pl.kernel(
        _sc_agg_body,
        out_type=(jax.ShapeDtypeStruct((N, D), jnp.float32),
                  jax.ShapeDtypeStruct((N, D), jnp.float32)),
        mesh=_mesh(),
        compiler_params=pltpu.CompilerParams(needs_layout_passes=False),
        scratch_types=[
            pltpu.VMEM((RING, CA), jnp.int32),
            pltpu.VMEM((RING, CA), jnp.int32),
            pltpu.VMEM((RING, CA, D), jnp.float32),
            pltpu.VMEM_SHARED((N, D), jnp.float32),
        ] + [pltpu.SemaphoreType.DMA] * (3 * RING),
    )


def _sc_agg(row, col, hh, t, zrows):
    return _sc_agg_kernel()(row, col, hh, t, zrows)


EB = 4000


def _tc_edge_mlp_body(ea_ref, w_ref, b_ref, t_ref):
    z = jnp.dot(ea_ref[...], w_ref[...], preferred_element_type=jnp.float32)
    t_ref[...] = jnp.maximum(z + b_ref[...][None, :], 0.0)


def _tc_edge_mlp(edge_attr, w, b):
    return pl.pallas_call(
        _tc_edge_mlp_body,
        grid=(E // EB,),
        in_specs=[
            pl.BlockSpec((EB, E_DIM), lambda i: (i, 0)),
            pl.BlockSpec((E_DIM, D), lambda i: (0, 0)),
            pl.BlockSpec((D,), lambda i: (0,)),
        ],
        out_specs=pl.BlockSpec((EB, D), lambda i: (i, 0)),
        out_shape=jax.ShapeDtypeStruct((E, D), jnp.float32),
    )(edge_attr, w, b)


def _tc_degred_body(dp_ref, deg_ref, dis_ref):
    deg = jnp.sum(dp_ref[...], axis=0)
    deg_ref[...] = deg
    dis_ref[...] = jnp.where(deg > 0, lax.rsqrt(jnp.maximum(deg, 1.0)), 0.0)


def _tc_degred(deg_part):
    return pl.pallas_call(
        _tc_degred_body,
        out_shape=[jax.ShapeDtypeStruct((1, N), jnp.float32),
                   jax.ShapeDtypeStruct((1, N), jnp.float32)],
    )(deg_part)


def _tc_prep_body(x_ref, w0_ref, dis_ref, hh_ref):
    h = jnp.dot(x_ref[...], w0_ref[...], preferred_element_type=jnp.float32)
    hh_ref[...] = dis_ref[...] * h


def _tc_prep(x, w0, dis_c):
    return pl.pallas_call(
        _tc_prep_body,
        out_shape=jax.ShapeDtypeStruct((N, D), jnp.float32),
    )(x, w0, dis_c)


def _layer_out(a1_ref, a2_ref, dis, deg, we2_ref, be2_ref, b_ref, g_ref, bt_ref):
    a1 = a1_ref[...]
    a2 = a2_ref[...]
    out = (dis * a1
           + jnp.dot(a2, we2_ref[...], preferred_element_type=jnp.float32)
           + deg * be2_ref[...][None, :]
           + b_ref[...][None, :])
    mu = jnp.mean(out, axis=-1, keepdims=True)
    var = jnp.mean((out - mu) ** 2, axis=-1, keepdims=True)
    out = (out - mu) / jnp.sqrt(var + EPS) * g_ref[...][None, :] + bt_ref[...][None, :]
    return jnp.maximum(out, 0.0)


def _tc_epi0_body(a1_ref, a2_ref, dis_ref, deg_ref, we2_ref, be2_ref, b_ref,
                  g_ref, bt_ref, w1_ref, hh_ref):
    dis = dis_ref[...]
    out = _layer_out(a1_ref, a2_ref, dis, deg_ref[...], we2_ref, be2_ref,
                     b_ref, g_ref, bt_ref)
    h1 = jnp.dot(out, w1_ref[...], preferred_element_type=jnp.float32)
    hh_ref[...] = dis * h1


def _tc_epi0(a1, a2, dis, deg, we2, be2, b, g, bt, w1):
    return pl.pallas_call(
        _tc_epi0_body,
        out_shape=jax.ShapeDtypeStruct((N, D), jnp.float32),
    )(a1, a2, dis, deg, we2, be2, b, g, bt, w1)


def _tc_epi1_body(a1_ref, a2_ref, dis_ref, deg_ref, we2_ref, be2_ref, b_ref,
                  g_ref, bt_ref, out_ref):
    out_ref[...] = _layer_out(a1_ref, a2_ref, dis_ref[...], deg_ref[...],
                              we2_ref, be2_ref, b_ref, g_ref, bt_ref)


def _tc_epi1(a1, a2, dis, deg, we2, be2, b, g, bt):
    return pl.pallas_call(
        _tc_epi1_body,
        out_shape=jax.ShapeDtypeStruct((N, D), jnp.float32),
    )(a1, a2, dis, deg, we2, be2, b, g, bt)


def kernel(x, edge_index, edge_attr, W0, We1_0, be1_0, We2_0, be2_0, b0, g0,
           bt0, W1, We1_1, be1_1, We2_1, be2_1, b1, g1, bt1):
    row = edge_index[0]
    col = edge_index[1]
    zeros_n = jnp.zeros((N,), jnp.float32)
    zrows = jnp.zeros((ROWS_LAST, D), jnp.float32)

    deg_part = _sc_deg(col, zeros_n)
    t0 = _tc_edge_mlp(edge_attr, We1_0, be1_0)
    t1 = _tc_edge_mlp(edge_attr, We1_1, be1_1)
    deg, dis = _tc_degred(deg_part)
    deg_c = deg.reshape(N, 1)
    dis_c = dis.reshape(N, 1)
    hh0 = _tc_prep(x, W0, dis_c)
    a1_0, a2_0 = _sc_agg(row, col, hh0, t0, zrows)
    hh1 = _tc_epi0(a1_0, a2_0, dis_c, deg_c, We2_0, be2_0, b0, g0, bt0, W1)
    a1_1, a2_1 = _sc_agg(row, col, hh1, t1, zrows)
    return _tc_epi1(a1_1, a2_1, dis_c, deg_c, We2_1, be2_1, b1, g1, bt1)


# speedup vs baseline: 1.0957x; 1.0957x over previous
import functools

import jax
import jax.numpy as jnp
from jax import lax
from jax.experimental import pallas as pl
from jax.experimental.pallas import tpu as pltpu
from jax.experimental.pallas import tpu_sc as plsc

N = 10000
E = 320000
D = 128
E_DIM = 16
EPS = 1e-5

NC = 2
NS = 16
NW = NC * NS

C = 128
CHUNKS = E // C

CA = 80
E2 = E // 2
TILE_EA = E2 // NS
NCHA = TILE_EA // CA
RING = 4

ROWS_A = 624
ROWS_LAST = N - (NS - 1) * ROWS_A

@functools.cache
def _mesh():
    return plsc.VectorSubcoreMesh(
        core_axis_name="c", subcore_axis_name="s", num_cores=NC, num_subcores=NS)


def _sc_deg_body(col_hbm, zeros_n, deg_out, cbuf, deg_local, sem):
    c = lax.axis_index("c")
    s = lax.axis_index("s")
    wid = s * NC + c
    pltpu.sync_copy(zeros_n, deg_local)
    nbase = CHUNKS // NW
    n_i = nbase + jnp.where(wid < CHUNKS % NW, 1, 0)
    ones = jnp.full((16,), 1.0, jnp.float32)

    @pl.loop(0, n_i)
    def _(i):
        base = (wid + i * NW) * C
        pltpu.sync_copy(col_hbm.at[pl.ds(base, C)], cbuf)
        for j in range(C // 16):
            idx = cbuf[pl.ds(j * 16, 16)]
            plsc.addupdate_scatter(deg_local, [idx], ones)

    pltpu.sync_copy(deg_local, deg_out.at[wid].at[0])


@functools.cache
def _sc_deg_kernel():
    return pl.kernel(
        _sc_deg_body,
        out_type=jax.ShapeDtypeStruct((NW, 1, N), jnp.float32),
        mesh=_mesh(),
        compiler_params=pltpu.CompilerParams(needs_layout_passes=False),
        scratch_types=[
            pltpu.VMEM((C,), jnp.int32),
            pltpu.VMEM((N,), jnp.float32),
            pltpu.SemaphoreType.DMA,
        ],
    )


def _sc_deg(col, zeros_n):
    return _sc_deg_kernel()(col, zeros_n)


def _zero_acc(z128, acc, r0, s):
    @pl.when(s < NS - 1)
    def _():
        pltpu.sync_copy(z128.at[pl.ds(0, ROWS_A)], acc.at[pl.ds(r0, ROWS_A)])

    @pl.when(s == NS - 1)
    def _():
        pltpu.sync_copy(z128, acc.at[pl.ds(r0, ROWS_LAST)])


def _read_acc(acc, out, c, r0, s):
    @pl.when(s < NS - 1)
    def _():
        pltpu.sync_copy(acc.at[pl.ds(r0, ROWS_A)],
                        out.at[c].at[pl.ds(r0, ROWS_A)])

    @pl.when(s == NS - 1)
    def _():
        pltpu.sync_copy(acc.at[pl.ds(r0, ROWS_LAST)],
                        out.at[c].at[pl.ds(r0, ROWS_LAST)])


def _sc_a1_body(row_hbm, col_hbm, hh_hbm, z128, a1p_out,
                rbuf, cbuf, gbuf, acc, *sems):
    c = lax.axis_index("c")
    s = lax.axis_index("s")
    r0 = s * ROWS_A
    seml = sems[:RING]
    sems_ = sems[RING:2 * RING]
    semg = sems[2 * RING:]
    base = c * E2 + s * TILE_EA

    def loads(j, b):
        off = base + j * CA
        pltpu.async_copy(col_hbm.at[pl.ds(off, CA)], cbuf.at[b], seml[b])
        pltpu.async_copy(row_hbm.at[pl.ds(off, CA)], rbuf.at[b], seml[b])

    def wait_loads(j, b):
        off = base + j * CA
        pltpu.make_async_copy(
            col_hbm.at[pl.ds(off, CA)], cbuf.at[b], seml[b]).wait()
        pltpu.make_async_copy(
            row_hbm.at[pl.ds(off, CA)], rbuf.at[b], seml[b]).wait()

    def gather(b):
        pltpu.async_copy(hh_hbm.at[rbuf.at[b]], gbuf.at[b], semg[b])

    def wait_gather(b):
        pltpu.make_async_copy(hh_hbm.at[rbuf.at[b]], gbuf.at[b], semg[b]).wait()

    def scatter(b):
        pltpu.async_copy(gbuf.at[b], acc.at[cbuf.at[b]], sems_[b], add=True)

    def wait_scatter(b):
        pltpu.make_async_copy(gbuf.at[b], acc.at[cbuf.at[b]], sems_[b]).wait()

    _zero_acc(z128, acc, r0, s)
    loads(0, 0)
    loads(1, 1)
    wait_loads(0, 0)
    gather(0)
    plsc.subcore_barrier()

    @pl.loop(0, (NCHA + RING - 1) // RING)
    def _(p):
        for q in range(RING):
            b = q
            bn = (q + 1) % RING
            b2 = (q + 2) % RING
            j = p * RING + q

            @pl.when(j < NCHA)
            def _():
                wait_gather(b)

                @pl.when(j >= 2)
                def _():
                    wait_scatter(b2)

                @pl.when(j + 2 < NCHA)
                def _():
                    loads(j + 2, b2)

                scatter(b)

                @pl.when(j + 1 < NCHA)
                def _():
                    wait_loads(j + 1, bn)
                    gather(bn)

    wait_scatter((NCHA - 2) % RING)
    wait_scatter((NCHA - 1) % RING)
    plsc.subcore_barrier()
    _read_acc(acc, a1p_out, c, r0, s)


def _sc_a2_body(col_hbm, t_hbm, z128, a2p_out, cbuf, tbuf, acc, *sems):
    c = lax.axis_index("c")
    s = lax.axis_index("s")
    r0 = s * ROWS_A
    seml = sems[:RING]
    sems_ = sems[RING:]
    base = c * E2 + s * TILE_EA

    def loads(j, b):
        off = base + j * CA
        pltpu.async_copy(col_hbm.at[pl.ds(off, CA)], cbuf.at[b], seml[b])
        pltpu.async_copy(t_hbm.at[pl.ds(off, CA)], tbuf.at[b], seml[b])

    def wait_loads(j, b):
        off = base + j * CA
        pltpu.make_async_copy(
            col_hbm.at[pl.ds(off, CA)], cbuf.at[b], seml[b]).wait()
        pltpu.make_async_copy(
            t_hbm.at[pl.ds(off, CA)], tbuf.at[b], seml[b]).wait()

    def scatter(b):
        pltpu.async_copy(tbuf.at[b], acc.at[cbuf.at[b]], sems_[b], add=True)

    def wait_scatter(b):
        pltpu.make_async_copy(tbuf.at[b], acc.at[cbuf.at[b]], sems_[b]).wait()

    _zero_acc(z128, acc, r0, s)
    loads(0, 0)
    loads(1, 1)
    plsc.subcore_barrier()

    @pl.loop(0, (NCHA + RING - 1) // RING)
    def _(p):
        for q in range(RING):
            b = q
            b2 = (q + 2) % RING
            j = p * RING + q

            @pl.when(j < NCHA)
            def _():
                wait_loads(j, b)

                @pl.when(j >= 2)
                def _():
                    wait_scatter(b2)

                @pl.when(j + 2 < NCHA)
                def _():
                    loads(j + 2, b2)

                scatter(b)

    wait_scatter((NCHA - 2) % RING)
    wait_scatter((NCHA - 1) % RING)
    plsc.subcore_barrier()
    _read_acc(acc, a2p_out, c, r0, s)


@functools.cache
def _sc_a1_kernel():
    return pl.kernel(
        _sc_a1_body,
        out_type=jax.ShapeDtypeStruct((NC, N, D), jnp.float32),
        mesh=_mesh(),
        compiler_params=pltpu.CompilerParams(needs_layout_passes=False),
        scratch_types=[
            pltpu.VMEM((RING, CA), jnp.int32),
            pltpu.VMEM((RING, CA), jnp.int32),
            pltpu.VMEM((RING, CA, D), jnp.float32),
            pltpu.VMEM_SHARED((N, D), jnp.float32),
        ] + [pltpu.SemaphoreType.DMA] * (3 * RING),
    )


@functools.cache
def _sc_a2_kernel():
    return pl.kernel(
        _sc_a2_body,
        out_type=jax.ShapeDtypeStruct((NC, N, D), jnp.float32),
        mesh=_mesh(),
        compiler_params=pltpu.CompilerParams(needs_layout_passes=False),
        scratch_types=[
            pltpu.VMEM((RING, CA), jnp.int32),
            pltpu.VMEM((RING, CA, D), jnp.float32),
            pltpu.VMEM_SHARED((N, D), jnp.float32),
        ] + [pltpu.SemaphoreType.DMA] * (2 * RING),
    )


def _sc_a1(row, col, hh, z128):
    return _sc_a1_kernel()(row, col, hh, z128)


def _sc_a2(col, t, z128):
    return _sc_a2_kernel()(col, t, z128)


EB = 4000


def _tc_edge_mlp_body(ea_ref, w_ref, b_ref, t_ref):
    z = jnp.dot(ea_ref[...], w_ref[...], preferred_element_type=jnp.float32)
    t_ref[...] = jnp.maximum(z + b_ref[...][None, :], 0.0)


def _tc_edge_mlp(edge_attr, w, b):
    return pl.pallas_call(
        _tc_edge_mlp_body,
        grid=(E // EB,),
        in_specs=[
            pl.BlockSpec((EB, E_DIM), lambda i: (i, 0)),
            pl.BlockSpec((E_DIM, D), lambda i: (0, 0)),
            pl.BlockSpec((D,), lambda i: (0,)),
        ],
        out_specs=pl.BlockSpec((EB, D), lambda i: (i, 0)),
        out_shape=jax.ShapeDtypeStruct((E, D), jnp.float32),
    )(edge_attr, w, b)


def _tc_degred_body(dp_ref, deg_ref, dis_ref):
    deg = jnp.sum(dp_ref[...], axis=0)
    deg_ref[...] = deg
    dis_ref[...] = jnp.where(deg > 0, lax.rsqrt(jnp.maximum(deg, 1.0)), 0.0)


def _tc_degred(deg_part):
    return pl.pallas_call(
        _tc_degred_body,
        out_shape=[jax.ShapeDtypeStruct((1, N), jnp.float32),
                   jax.ShapeDtypeStruct((1, N), jnp.float32)],
    )(deg_part)


def _tc_h_body(x_ref, w0_ref, h_ref):
    h_ref[...] = jnp.dot(x_ref[...], w0_ref[...],
                         preferred_element_type=jnp.float32)


def _tc_h(x, w0):
    return pl.pallas_call(
        _tc_h_body,
        out_shape=jax.ShapeDtypeStruct((N, D), jnp.float32),
    )(x, w0)


def _tc_scale_body(h_ref, dis_ref, hh_ref):
    hh_ref[...] = dis_ref[...] * h_ref[...]


def _tc_scale(h, dis_c):
    return pl.pallas_call(
        _tc_scale_body,
        out_shape=jax.ShapeDtypeStruct((N, D), jnp.float32),
    )(h, dis_c)


def _layer_out(a1p_ref, a2p_ref, dis, deg, we2_ref, be2_ref, b_ref, g_ref, bt_ref):
    a1 = a1p_ref[0] + a1p_ref[1]
    a2 = a2p_ref[0] + a2p_ref[1]
    out = (dis * a1
           + jnp.dot(a2, we2_ref[...], preferred_element_type=jnp.float32)
           + deg * be2_ref[...][None, :]
           + b_ref[...][None, :])
    mu = jnp.mean(out, axis=-1, keepdims=True)
    var = jnp.mean((out - mu) ** 2, axis=-1, keepdims=True)
    out = (out - mu) / jnp.sqrt(var + EPS) * g_ref[...][None, :] + bt_ref[...][None, :]
    return jnp.maximum(out, 0.0)


def _tc_epi0_body(a1p_ref, a2p_ref, dis_ref, deg_ref, we2_ref, be2_ref, b_ref,
                  g_ref, bt_ref, w1_ref, hh_ref):
    dis = dis_ref[...]
    out = _layer_out(a1p_ref, a2p_ref, dis, deg_ref[...], we2_ref, be2_ref,
                     b_ref, g_ref, bt_ref)
    h1 = jnp.dot(out, w1_ref[...], preferred_element_type=jnp.float32)
    hh_ref[...] = dis * h1


def _tc_epi0(a1p, a2p, dis, deg, we2, be2, b, g, bt, w1):
    return pl.pallas_call(
        _tc_epi0_body,
        out_shape=jax.ShapeDtypeStruct((N, D), jnp.float32),
    )(a1p, a2p, dis, deg, we2, be2, b, g, bt, w1)


def _tc_epi1_body(a1p_ref, a2p_ref, dis_ref, deg_ref, we2_ref, be2_ref, b_ref,
                  g_ref, bt_ref, out_ref):
    out_ref[...] = _layer_out(a1p_ref, a2p_ref, dis_ref[...], deg_ref[...],
                              we2_ref, be2_ref, b_ref, g_ref, bt_ref)


def _tc_epi1(a1p, a2p, dis, deg, we2, be2, b, g, bt):
    return pl.pallas_call(
        _tc_epi1_body,
        out_shape=jax.ShapeDtypeStruct((N, D), jnp.float32),
    )(a1p, a2p, dis, deg, we2, be2, b, g, bt)


def kernel(x, edge_index, edge_attr, W0, We1_0, be1_0, We2_0, be2_0, b0, g0,
           bt0, W1, We1_1, be1_1, We2_1, be2_1, b1, g1, bt1):
    row = edge_index[0]
    col = edge_index[1]
    zeros_n = jnp.zeros((N,), jnp.float32)
    z128 = jnp.zeros((ROWS_LAST, D), jnp.float32)

    deg_part = _sc_deg(col, zeros_n)
    t0 = _tc_edge_mlp(edge_attr, We1_0, be1_0)
    t1 = _tc_edge_mlp(edge_attr, We1_1, be1_1)
    h0 = _tc_h(x, W0)
    deg, dis = _tc_degred(deg_part)
    deg_c = deg.reshape(N, 1)
    dis_c = dis.reshape(N, 1)
    hh0 = _tc_scale(h0, dis_c)
    a2p0 = _sc_a2(col, t0, z128)
    a1p0 = _sc_a1(row, col, hh0, z128)
    a2p1 = _sc_a2(col, t1, z128)
    hh1 = _tc_epi0(a1p0, a2p0, dis_c, deg_c, We2_0, be2_0, b0, g0, bt0, W1)
    a1p1 = _sc_a1(row, col, hh1, z128)
    return _tc_epi1(a1p1, a2p1, dis_c, deg_c, We2_1, be2_1, b1, g1, bt1)


# speedup vs baseline: 1.2576x; 1.1478x over previous
import functools

import jax
import jax.numpy as jnp
from jax import lax
from jax.experimental import pallas as pl
from jax.experimental.pallas import tpu as pltpu
from jax.experimental.pallas import tpu_sc as plsc

N = 10000
E = 320000
D = 128
E_DIM = 16
EPS = 1e-5

NC = 2
NS = 16
NW = NC * NS

C = 512
CHUNKS = E // C

CA = 80
E2 = E // 2
TILE_EA = E2 // NS
NCHA = TILE_EA // CA
RING = 4
G1 = 4
L1 = 8

ROWS_A = 624
ROWS_LAST = N - (NS - 1) * ROWS_A

@functools.cache
def _mesh():
    return plsc.VectorSubcoreMesh(
        core_axis_name="c", subcore_axis_name="s", num_cores=NC, num_subcores=NS)


def _sc_deg_body(col_hbm, zeros_n, deg_out, cbuf, deg_local, *sems):
    c = lax.axis_index("c")
    s = lax.axis_index("s")
    wid = s * NC + c
    pltpu.sync_copy(zeros_n, deg_local)
    nbase = CHUNKS // NW
    n_i = nbase + jnp.where(wid < CHUNKS % NW, 1, 0)
    ones = jnp.full((16,), 1.0, jnp.float32)
    nmax = nbase + (1 if CHUNKS % NW else 0)

    def issue(i, b):
        base = (wid + i * NW) * C
        pltpu.async_copy(col_hbm.at[pl.ds(base, C)], cbuf.at[b], sems[b])

    def wait(i, b):
        base = (wid + i * NW) * C
        pltpu.make_async_copy(
            col_hbm.at[pl.ds(base, C)], cbuf.at[b], sems[b]).wait()

    @pl.when(n_i > 0)
    def _():
        issue(0, 0)

    @pl.when(n_i > 1)
    def _():
        issue(1, 1)

    @pl.loop(0, (nmax + 1) // 2)
    def _(p):
        for q in range(2):
            b = q
            i = p * 2 + q

            @pl.when(i < n_i)
            def _():
                wait(i, b)

                @pl.when(i + 2 < n_i)
                def _():
                    issue(i + 2, b)

                for j in range(C // 16):
                    idx = cbuf[b, pl.ds(j * 16, 16)]
                    plsc.addupdate_scatter(deg_local, [idx], ones)

    pltpu.sync_copy(deg_local, deg_out.at[wid].at[0])


@functools.cache
def _sc_deg_kernel():
    return pl.kernel(
        _sc_deg_body,
        out_type=jax.ShapeDtypeStruct((NW, 1, N), jnp.float32),
        mesh=_mesh(),
        compiler_params=pltpu.CompilerParams(needs_layout_passes=False),
        scratch_types=[
            pltpu.VMEM((2, C), jnp.int32),
            pltpu.VMEM((N,), jnp.float32),
            pltpu.SemaphoreType.DMA,
            pltpu.SemaphoreType.DMA,
        ],
    )


def _sc_deg(col, zeros_n):
    return _sc_deg_kernel()(col, zeros_n)


def _zero_acc(z128, acc, r0, s):
    @pl.when(s < NS - 1)
    def _():
        pltpu.sync_copy(z128.at[pl.ds(0, ROWS_A)], acc.at[pl.ds(r0, ROWS_A)])

    @pl.when(s == NS - 1)
    def _():
        pltpu.sync_copy(z128, acc.at[pl.ds(r0, ROWS_LAST)])


def _read_acc(acc, out, c, r0, s):
    @pl.when(s < NS - 1)
    def _():
        pltpu.sync_copy(acc.at[pl.ds(r0, ROWS_A)],
                        out.at[c].at[pl.ds(r0, ROWS_A)])

    @pl.when(s == NS - 1)
    def _():
        pltpu.sync_copy(acc.at[pl.ds(r0, ROWS_LAST)],
                        out.at[c].at[pl.ds(r0, ROWS_LAST)])


def _sc_a1_body(row_hbm, col_hbm, hh_hbm, z128, a1p_out,
                rbuf, cbuf, gbuf, acc, *sems):
    c = lax.axis_index("c")
    s = lax.axis_index("s")
    r0 = s * ROWS_A
    seml = sems[:L1]
    sems_ = sems[L1:L1 + G1]
    semg = sems[L1 + G1:]
    base = c * E2 + s * TILE_EA

    def loads(j, b):
        off = base + j * CA
        pltpu.async_copy(col_hbm.at[pl.ds(off, CA)], cbuf.at[b], seml[b])
        pltpu.async_copy(row_hbm.at[pl.ds(off, CA)], rbuf.at[b], seml[b])

    def wait_loads(j, b):
        off = base + j * CA
        pltpu.make_async_copy(
            col_hbm.at[pl.ds(off, CA)], cbuf.at[b], seml[b]).wait()
        pltpu.make_async_copy(
            row_hbm.at[pl.ds(off, CA)], rbuf.at[b], seml[b]).wait()

    def gather(bl, bg):
        pltpu.async_copy(hh_hbm.at[rbuf.at[bl]], gbuf.at[bg], semg[bg])

    def wait_gather(bl, bg):
        pltpu.make_async_copy(
            hh_hbm.at[rbuf.at[bl]], gbuf.at[bg], semg[bg]).wait()

    def scatter(bl, bg):
        pltpu.async_copy(gbuf.at[bg], acc.at[cbuf.at[bl]], sems_[bg], add=True)

    def wait_scatter(bl, bg):
        pltpu.make_async_copy(
            gbuf.at[bg], acc.at[cbuf.at[bl]], sems_[bg]).wait()

    _zero_acc(z128, acc, r0, s)
    loads(0, 0)
    loads(1, 1)
    loads(2, 2)
    wait_loads(0, 0)
    gather(0, 0)
    wait_loads(1, 1)
    gather(1, 1)
    plsc.subcore_barrier()

    @pl.loop(0, (NCHA + L1 - 1) // L1)
    def _(p):
        for q in range(L1):
            j = p * L1 + q
            bl = q
            bg = q % G1
            bl2 = (q + 2) % L1
            bg2 = (q + 2) % G1
            bl3 = (q + 3) % L1
            blp2 = (q - 2) % L1
            bgp2 = (q - 2) % G1

            @pl.when(j < NCHA)
            def _():
                wait_gather(bl, bg)

                @pl.when(j >= 2)
                def _():
                    wait_scatter(blp2, bgp2)

                @pl.when(j + 3 < NCHA)
                def _():
                    loads(j + 3, bl3)

                scatter(bl, bg)

                @pl.when(j + 2 < NCHA)
                def _():
                    wait_loads(j + 2, bl2)
                    gather(bl2, bg2)

    wait_scatter((NCHA - 2) % L1, (NCHA - 2) % G1)
    wait_scatter((NCHA - 1) % L1, (NCHA - 1) % G1)
    plsc.subcore_barrier()
    _read_acc(acc, a1p_out, c, r0, s)


def _sc_a2_body(col_hbm, t_hbm, z128, a2p_out, cbuf, tbuf, acc, *sems):
    c = lax.axis_index("c")
    s = lax.axis_index("s")
    r0 = s * ROWS_A
    seml = sems[:RING]
    sems_ = sems[RING:]
    base = c * E2 + s * TILE_EA

    def loads(j, b):
        off = base + j * CA
        pltpu.async_copy(col_hbm.at[pl.ds(off, CA)], cbuf.at[b], seml[b])
        pltpu.async_copy(t_hbm.at[pl.ds(off, CA)], tbuf.at[b], seml[b])

    def wait_loads(j, b):
        off = base + j * CA
        pltpu.make_async_copy(
            col_hbm.at[pl.ds(off, CA)], cbuf.at[b], seml[b]).wait()
        pltpu.make_async_copy(
            t_hbm.at[pl.ds(off, CA)], tbuf.at[b], seml[b]).wait()

    def scatter(b):
        pltpu.async_copy(tbuf.at[b], acc.at[cbuf.at[b]], sems_[b], add=True)

    def wait_scatter(b):
        pltpu.make_async_copy(tbuf.at[b], acc.at[cbuf.at[b]], sems_[b]).wait()

    _zero_acc(z128, acc, r0, s)
    loads(0, 0)
    loads(1, 1)
    plsc.subcore_barrier()

    @pl.loop(0, (NCHA + RING - 1) // RING)
    def _(p):
        for q in range(RING):
            b = q
            b2 = (q + 2) % RING
            j = p * RING + q

            @pl.when(j < NCHA)
            def _():
                wait_loads(j, b)

                @pl.when(j >= 2)
                def _():
                    wait_scatter(b2)

                @pl.when(j + 2 < NCHA)
                def _():
                    loads(j + 2, b2)

                scatter(b)

    wait_scatter((NCHA - 2) % RING)
    wait_scatter((NCHA - 1) % RING)
    plsc.subcore_barrier()
    _read_acc(acc, a2p_out, c, r0, s)


@functools.cache
def _sc_a1_kernel():
    return pl.kernel(
        _sc_a1_body,
        out_type=jax.ShapeDtypeStruct((NC, N, D), jnp.float32),
        mesh=_mesh(),
        compiler_params=pltpu.CompilerParams(needs_layout_passes=False),
        scratch_types=[
            pltpu.VMEM((L1, CA), jnp.int32),
            pltpu.VMEM((L1, CA), jnp.int32),
            pltpu.VMEM((G1, CA, D), jnp.float32),
            pltpu.VMEM_SHARED((N, D), jnp.float32),
        ] + [pltpu.SemaphoreType.DMA] * (L1 + 2 * G1),
    )


@functools.cache
def _sc_a2_kernel():
    return pl.kernel(
        _sc_a2_body,
        out_type=jax.ShapeDtypeStruct((NC, N, D), jnp.float32),
        mesh=_mesh(),
        compiler_params=pltpu.CompilerParams(needs_layout_passes=False),
        scratch_types=[
            pltpu.VMEM((RING, CA), jnp.int32),
            pltpu.VMEM((RING, CA, D), jnp.float32),
            pltpu.VMEM_SHARED((N, D), jnp.float32),
        ] + [pltpu.SemaphoreType.DMA] * (2 * RING),
    )


def _sc_a1(row, col, hh, z128):
    return _sc_a1_kernel()(row, col, hh, z128)


def _sc_a2(col, t, z128):
    return _sc_a2_kernel()(col, t, z128)


EB = 4000


def _tc_edge_mlp_body(ea_ref, w_ref, b_ref, t_ref):
    z = jnp.dot(ea_ref[...], w_ref[...], preferred_element_type=jnp.float32)
    t_ref[...] = jnp.maximum(z + b_ref[...][None, :], 0.0)


def _tc_edge_mlp(edge_attr, w, b):
    return pl.pallas_call(
        _tc_edge_mlp_body,
        grid=(E // EB,),
        in_specs=[
            pl.BlockSpec((EB, E_DIM), lambda i: (i, 0)),
            pl.BlockSpec((E_DIM, D), lambda i: (0, 0)),
            pl.BlockSpec((D,), lambda i: (0,)),
        ],
        out_specs=pl.BlockSpec((EB, D), lambda i: (i, 0)),
        out_shape=jax.ShapeDtypeStruct((E, D), jnp.float32),
    )(edge_attr, w, b)


def _tc_degred_body(dp_ref, deg_ref, dis_ref):
    deg = jnp.sum(dp_ref[...], axis=0)
    deg_ref[...] = deg
    dis_ref[...] = jnp.where(deg > 0, lax.rsqrt(jnp.maximum(deg, 1.0)), 0.0)


def _tc_degred(deg_part):
    return pl.pallas_call(
        _tc_degred_body,
        out_shape=[jax.ShapeDtypeStruct((1, N), jnp.float32),
                   jax.ShapeDtypeStruct((1, N), jnp.float32)],
    )(deg_part)


def _tc_h_body(x_ref, w0_ref, h_ref):
    h_ref[...] = jnp.dot(x_ref[...], w0_ref[...],
                         preferred_element_type=jnp.float32)


def _tc_h(x, w0):
    return pl.pallas_call(
        _tc_h_body,
        out_shape=jax.ShapeDtypeStruct((N, D), jnp.float32),
    )(x, w0)


def _tc_scale_body(h_ref, dis_ref, hh_ref):
    hh_ref[...] = dis_ref[...] * h_ref[...]


def _tc_scale(h, dis_c):
    return pl.pallas_call(
        _tc_scale_body,
        out_shape=jax.ShapeDtypeStruct((N, D), jnp.float32),
    )(h, dis_c)


def _layer_out(a1p_ref, a2p_ref, dis, deg, we2_ref, be2_ref, b_ref, g_ref, bt_ref):
    a1 = a1p_ref[0] + a1p_ref[1]
    a2 = a2p_ref[0] + a2p_ref[1]
    out = (dis * a1
           + jnp.dot(a2, we2_ref[...], preferred_element_type=jnp.float32)
           + deg * be2_ref[...][None, :]
           + b_ref[...][None, :])
    mu = jnp.mean(out, axis=-1, keepdims=True)
    var = jnp.mean((out - mu) ** 2, axis=-1, keepdims=True)
    out = (out - mu) / jnp.sqrt(var + EPS) * g_ref[...][None, :] + bt_ref[...][None, :]
    return jnp.maximum(out, 0.0)


def _tc_epi0_body(a1p_ref, a2p_ref, dis_ref, deg_ref, we2_ref, be2_ref, b_ref,
                  g_ref, bt_ref, w1_ref, hh_ref):
    dis = dis_ref[...]
    out = _layer_out(a1p_ref, a2p_ref, dis, deg_ref[...], we2_ref, be2_ref,
                     b_ref, g_ref, bt_ref)
    h1 = jnp.dot(out, w1_ref[...], preferred_element_type=jnp.float32)
    hh_ref[...] = dis * h1


def _tc_epi0(a1p, a2p, dis, deg, we2, be2, b, g, bt, w1):
    return pl.pallas_call(
        _tc_epi0_body,
        out_shape=jax.ShapeDtypeStruct((N, D), jnp.float32),
    )(a1p, a2p, dis, deg, we2, be2, b, g, bt, w1)


def _tc_epi1_body(a1p_ref, a2p_ref, dis_ref, deg_ref, we2_ref, be2_ref, b_ref,
                  g_ref, bt_ref, out_ref):
    out_ref[...] = _layer_out(a1p_ref, a2p_ref, dis_ref[...], deg_ref[...],
                              we2_ref, be2_ref, b_ref, g_ref, bt_ref)


def _tc_epi1(a1p, a2p, dis, deg, we2, be2, b, g, bt):
    return pl.pallas_call(
        _tc_epi1_body,
        out_shape=jax.ShapeDtypeStruct((N, D), jnp.float32),
    )(a1p, a2p, dis, deg, we2, be2, b, g, bt)


def kernel(x, edge_index, edge_attr, W0, We1_0, be1_0, We2_0, be2_0, b0, g0,
           bt0, W1, We1_1, be1_1, We2_1, be2_1, b1, g1, bt1):
    row = edge_index[0]
    col = edge_index[1]
    zeros_n = jnp.zeros((N,), jnp.float32)
    z128 = jnp.zeros((ROWS_LAST, D), jnp.float32)

    deg_part = _sc_deg(col, zeros_n)
    t0 = _tc_edge_mlp(edge_attr, We1_0, be1_0)
    t1 = _tc_edge_mlp(edge_attr, We1_1, be1_1)
    h0 = _tc_h(x, W0)
    deg, dis = _tc_degred(deg_part)
    deg_c = deg.reshape(N, 1)
    dis_c = dis.reshape(N, 1)
    hh0 = _tc_scale(h0, dis_c)
    a2p0 = _sc_a2(col, t0, z128)
    a1p0 = _sc_a1(row, col, hh0, z128)
    a2p1 = _sc_a2(col, t1, z128)
    hh1 = _tc_epi0(a1p0, a2p0, dis_c, deg_c, We2_0, be2_0, b0, g0, bt0, W1)
    a1p1 = _sc_a1(row, col, hh1, z128)
    return _tc_epi1(a1p1, a2p1, dis_c, deg_c, We2_1, be2_1, b1, g1, bt1)
